# Initial kernel scaffold; baseline (speedup 1.0000x reference)
import jax
import jax.numpy as jnp
from jax.experimental import pallas as pl


def kernel(x, params):
    raise NotImplementedError("write your pallas kernel here")


# speedup vs baseline: 7.7591x; 7.7591x over previous
import functools

import jax
import jax.numpy as jnp
from jax import lax
from jax.experimental import pallas as pl
from jax.experimental.pallas import tpu as pltpu
from jax.experimental.pallas import tpu_sc as plsc

_B = 16
_EPS = 1e-5
_HI = lax.Precision.HIGHEST


def _fps(coords, npoint, interpret=False):
    Bb, _, N = coords.shape

    def body(c_ref, o_ref):
        xc = c_ref[:, 0, :]
        yc = c_ref[:, 1, :]
        zc = c_ref[:, 2, :]
        lane = lax.broadcasted_iota(jnp.int32, (Bb, N), 1)

        def step(j, carry):
            dist, far, sel = carry
            pick = (lane == far)
            pi = pick.astype(jnp.int32)
            sel = sel * (1 - pi) + j * pi
            oh = pick.astype(jnp.float32)
            cx = jnp.sum(xc * oh, axis=1, keepdims=True)
            cy = jnp.sum(yc * oh, axis=1, keepdims=True)
            cz = jnp.sum(zc * oh, axis=1, keepdims=True)
            dx = xc - cx
            dy = yc - cy
            dz = zc - cz
            d = dx * dx + dy * dy + dz * dz
            dist = jnp.minimum(dist, d)
            m = jnp.max(dist, axis=1, keepdims=True)
            far = jnp.min(jnp.where(dist == m, lane, N), axis=1, keepdims=True)
            return dist, far, sel

        dist0 = xc * 0.0 + 1e10
        far0 = jnp.zeros((Bb, 1), jnp.int32)
        sel0 = (xc * 0.0).astype(jnp.int32) + 2 * N
        _, _, sel = lax.fori_loop(0, npoint, step, (dist0, far0, sel0))
        ii = lax.broadcasted_iota(jnp.int32, (1, N), 1).astype(jnp.float32)
        oiota = lax.broadcasted_iota(jnp.int32, (N, npoint), 1)
        rows = []
        for b in range(Bb):
            ohb = (sel[b][:, None] == oiota).astype(jnp.float32)
            rows.append(lax.dot_general(ii, ohb, (((1,), (0,)), ((), ())),
                                        precision=_HI))
        o_ref[...] = jnp.concatenate(rows, axis=0).astype(jnp.int32)

    return pl.pallas_call(
        body,
        out_shape=jax.ShapeDtypeStruct((Bb, npoint), jnp.int32),
        interpret=interpret,
    )(coords)


def _cgather(coords, idx, interpret=False):
    Bb, _, N = coords.shape
    S = idx.shape[1]

    def body(c_ref, i_ref, o_ref):
        g = i_ref[0, 0]
        oh = (lax.broadcasted_iota(jnp.int32, (N, S), 0) == g[None, :]).astype(
            jnp.float32)
        c = c_ref[0]
        o_ref[0] = lax.dot_general(c, oh, (((1,), (0,)), ((), ())),
                                   precision=_HI)

    return pl.pallas_call(
        body,
        grid=(Bb,),
        in_specs=[pl.BlockSpec((1, 3, N), lambda b: (b, 0, 0)),
                  pl.BlockSpec((1, 1, S), lambda b: (b, 0, 0))],
        out_specs=pl.BlockSpec((1, 3, S), lambda b: (b, 0, 0)),
        out_shape=jax.ShapeDtypeStruct((Bb, 3, S), jnp.float32),
        interpret=interpret,
    )(coords, idx.reshape(Bb, 1, S))


def _ball(coords, centers, r2, K, interpret=False):
    Bb, _, N = coords.shape
    S = centers.shape[2]

    def body(c_ref, s_ref, o_ref):
        b = pl.program_id(0)
        x0, x1, x2 = c_ref[0, 0], c_ref[0, 1], c_ref[0, 2]
        s0, s1, s2 = s_ref[0, 0], s_ref[0, 1], s_ref[0, 2]
        xsq = x0 * x0 + x1 * x1 + x2 * x2
        ssq = s0 * s0 + s1 * s1 + s2 * s2
        dot = lax.dot_general(s_ref[0], c_ref[0],
                              (((0,), (0,)), ((), ())))
        sqrd = (ssq[:, None] - 2.0 * dot) + xsq[None, :]
        mask = sqrd <= r2
        maskf = mask.astype(jnp.float32)
        lt = (lax.broadcasted_iota(jnp.int32, (N, N), 0)
              <= lax.broadcasted_iota(jnp.int32, (N, N), 1)).astype(jnp.float32)
        cnt = lax.dot_general(maskf, lt, (((1,), (0,)), ((), ())),
                              precision=_HI)
        rank = cnt * maskf
        lane = lax.broadcasted_iota(jnp.int32, (S, N), 1)
        cols = []
        for k in range(K):
            cand = jnp.where(mask & (rank == float(k + 1)), lane, N)
            cols.append(jnp.min(cand, axis=1))
        first = cols[0]
        off = b * N
        for k in range(K):
            gk = jnp.where(cols[k] == N, first, cols[k]) + off
            o_ref[0, k, :] = gk

    return pl.pallas_call(
        body,
        grid=(Bb,),
        in_specs=[pl.BlockSpec((1, 3, N), lambda b: (b, 0, 0)),
                  pl.BlockSpec((1, 3, S), lambda b: (b, 0, 0))],
        out_specs=pl.BlockSpec((1, K, S), lambda b: (b, 0, 0)),
        out_shape=jax.ShapeDtypeStruct((Bb, K, S), jnp.int32),
        interpret=interpret,
    )(coords, centers)


def _sc_gather(table, idx):
    M, = idx.shape
    R, C = table.shape
    NW = 32
    per_w = M // NW
    chunk = min(128, per_w)
    n_it = per_w // chunk
    mesh = plsc.VectorSubcoreMesh(core_axis_name="c", subcore_axis_name="s")

    @functools.partial(
        pl.kernel, mesh=mesh,
        out_type=jax.ShapeDtypeStruct((M, C), jnp.float32),
        compiler_params=pltpu.CompilerParams(use_tc_tiling_on_sc=False),
        scratch_types=[pltpu.VMEM((chunk,), jnp.int32),
                       pltpu.VMEM((chunk, C), jnp.float32),
                       pltpu.SemaphoreType.DMA])
    def k(table_hbm, idx_hbm, out_hbm, idx_v, rows_v, sem):
        wid = lax.axis_index("s") * 2 + lax.axis_index("c")
        base = wid * per_w

        def it(i, carry):
            off = base + i * chunk
            pltpu.sync_copy(idx_hbm.at[pl.ds(off, chunk)], idx_v)
            pltpu.async_copy(table_hbm.at[idx_v], rows_v, sem).wait()
            pltpu.sync_copy(rows_v, out_hbm.at[pl.ds(off, chunk)])
            return carry

        lax.fori_loop(0, n_it, it, 0)

    return k(table, idx)


def _y1(g, ct, wx_ref, wf_ref, b_ref, CF):
    if CF:
        gx = g[:, :, CF:CF + 3] - ct[:, None, :]
        y = (lax.dot_general(gx, wx_ref[...], (((2,), (0,)), ((), ())))
             + lax.dot_general(g[:, :, :CF], wf_ref[...],
                               (((2,), (0,)), ((), ()))))
    else:
        gx = g[:, :, :3] - ct[:, None, :]
        y = lax.dot_general(gx, wx_ref[...], (((2,), (0,)), ((), ())))
    return y + b_ref[0][None, None, :]


def _stats(G, CT, Wx, Wf, b, interpret=False):
    BS, K, CP = G.shape
    C1 = Wx.shape[1]
    TS = 2048 // K
    grid = BS // TS
    CF = 0 if Wf is None else Wf.shape[0]

    def body(g_ref, ct_ref, wx_ref, wf_ref, b_ref, o_ref):
        y = _y1(g_ref[...], ct_ref[...], wx_ref, wf_ref, b_ref, CF)
        s0 = jnp.sum(y, axis=(0, 1))
        s1 = jnp.sum(y * y, axis=(0, 1))
        part = jnp.concatenate(
            [s0[None], s1[None], jnp.zeros((6, C1), jnp.float32)], axis=0)

        @pl.when(pl.program_id(0) == 0)
        def _():
            o_ref[...] = part

        @pl.when(pl.program_id(0) > 0)
        def _():
            o_ref[...] += part

    wf = Wx if Wf is None else Wf
    return pl.pallas_call(
        body,
        grid=(grid,),
        in_specs=[pl.BlockSpec((TS, K, CP), lambda t: (t, 0, 0)),
                  pl.BlockSpec((TS, 3), lambda t: (t, 0)),
                  pl.BlockSpec(Wx.shape, lambda t: (0, 0)),
                  pl.BlockSpec(wf.shape, lambda t: (0, 0)),
                  pl.BlockSpec((1, C1), lambda t: (0, 0))],
        out_specs=pl.BlockSpec((8, C1), lambda t: (0, 0)),
        out_shape=jax.ShapeDtypeStruct((8, C1), jnp.float32),
        interpret=interpret,
    )(G, CT, Wx, wf, b)


def _mid(G, CT, Wx, Wf, b, a1, c1, W2, b2, interpret=False):
    BS, K, CP = G.shape
    C1 = Wx.shape[1]
    C2 = W2.shape[1]
    TS = 2048 // K
    grid = BS // TS
    CF = 0 if Wf is None else Wf.shape[0]

    def body(g_ref, ct_ref, wx_ref, wf_ref, b_ref, a_ref, c_ref, w_ref,
             b2_ref, y_ref, s_ref):
        y1 = _y1(g_ref[...], ct_ref[...], wx_ref, wf_ref, b_ref, CF)
        x = jnp.maximum(y1 * a_ref[0][None, None, :] + c_ref[0][None, None, :],
                        0.0)
        y2 = lax.dot_general(x, w_ref[...], (((2,), (0,)), ((), ())),
                             preferred_element_type=jnp.float32)
        y2 = y2 + b2_ref[0][None, None, :]
        y_ref[...] = y2
        s0 = jnp.sum(y2, axis=(0, 1))
        s1 = jnp.sum(y2 * y2, axis=(0, 1))
        part = jnp.concatenate(
            [s0[None], s1[None], jnp.zeros((6, C2), jnp.float32)], axis=0)

        @pl.when(pl.program_id(0) == 0)
        def _():
            s_ref[...] = part

        @pl.when(pl.program_id(0) > 0)
        def _():
            s_ref[...] += part

    wf = Wx if Wf is None else Wf
    return pl.pallas_call(
        body,
        grid=(grid,),
        in_specs=[pl.BlockSpec((TS, K, CP), lambda t: (t, 0, 0)),
                  pl.BlockSpec((TS, 3), lambda t: (t, 0)),
                  pl.BlockSpec(Wx.shape, lambda t: (0, 0)),
                  pl.BlockSpec(wf.shape, lambda t: (0, 0)),
                  pl.BlockSpec((1, C1), lambda t: (0, 0)),
                  pl.BlockSpec((1, C1), lambda t: (0, 0)),
                  pl.BlockSpec((1, C1), lambda t: (0, 0)),
                  pl.BlockSpec((C1, C2), lambda t: (0, 0)),
                  pl.BlockSpec((1, C2), lambda t: (0, 0))],
        out_specs=[pl.BlockSpec((TS, K, C2), lambda t: (t, 0, 0)),
                   pl.BlockSpec((8, C2), lambda t: (0, 0))],
        out_shape=[jax.ShapeDtypeStruct((BS, K, C2), jnp.float32),
                   jax.ShapeDtypeStruct((8, C2), jnp.float32)],
        interpret=interpret,
    )(G, CT, Wx, wf, b, a1, c1, W2, b2)


def _lastpool(y2, a2, c2, W3, b3, interpret=False):
    BS, K, C2 = y2.shape
    C3 = W3.shape[1]
    TS = 2048 // K
    grid = BS // TS

    def body(y_ref, a_ref, c_ref, w_ref, b_ref, p_ref, s_ref):
        x = jnp.maximum(
            y_ref[...] * a_ref[0][None, None, :] + c_ref[0][None, None, :], 0.0)
        y3 = lax.dot_general(x, w_ref[...], (((2,), (0,)), ((), ())),
                             preferred_element_type=jnp.float32)
        y3 = y3 + b_ref[0][None, None, :]
        p_ref[...] = jnp.max(y3, axis=1)
        s0 = jnp.sum(y3, axis=(0, 1))
        s1 = jnp.sum(y3 * y3, axis=(0, 1))
        part = jnp.concatenate(
            [s0[None], s1[None], jnp.zeros((6, C3), jnp.float32)], axis=0)

        @pl.when(pl.program_id(0) == 0)
        def _():
            s_ref[...] = part

        @pl.when(pl.program_id(0) > 0)
        def _():
            s_ref[...] += part

    return pl.pallas_call(
        body,
        grid=(grid,),
        in_specs=[pl.BlockSpec((TS, K, C2), lambda t: (t, 0, 0)),
                  pl.BlockSpec((1, C2), lambda t: (0, 0)),
                  pl.BlockSpec((1, C2), lambda t: (0, 0)),
                  pl.BlockSpec((C2, C3), lambda t: (0, 0)),
                  pl.BlockSpec((1, C3), lambda t: (0, 0))],
        out_specs=[pl.BlockSpec((TS, C3), lambda t: (t, 0)),
                   pl.BlockSpec((8, C3), lambda t: (0, 0))],
        out_shape=[jax.ShapeDtypeStruct((BS, C3), jnp.float32),
                   jax.ShapeDtypeStruct((8, C3), jnp.float32)],
        interpret=interpret,
    )(y2, a2, c2, W3, b3)


def _table2(c1, P1, a3, c3, interpret=False):
    Bb, _, S = c1.shape
    C1 = P1.shape[2]

    def body(c_ref, p_ref, a_ref, cc_ref, o_ref):
        pts = jnp.maximum(p_ref[0] * a_ref[0][None, :] + cc_ref[0][None, :],
                          0.0)
        o_ref[0, :, :C1] = pts
        eye = (lax.broadcasted_iota(jnp.int32, (3, 16), 0)
               == lax.broadcasted_iota(jnp.int32, (3, 16), 1)).astype(
                   jnp.float32)
        o_ref[0, :, C1:] = lax.dot_general(
            c_ref[0], eye, (((0,), (0,)), ((), ())), precision=_HI)

    return pl.pallas_call(
        body,
        grid=(Bb,),
        in_specs=[pl.BlockSpec((1, 3, S), lambda b: (b, 0, 0)),
                  pl.BlockSpec((1, S, C1), lambda b: (b, 0, 0)),
                  pl.BlockSpec((1, C1), lambda b: (0, 0)),
                  pl.BlockSpec((1, C1), lambda b: (0, 0))],
        out_specs=pl.BlockSpec((1, S, C1 + 16), lambda b: (b, 0, 0)),
        out_shape=jax.ShapeDtypeStruct((Bb, S, C1 + 16), jnp.float32),
        interpret=interpret,
    )(c1, P1, a3, c3)


def _sa3_head(xyzf, P2, a6, c6, w, interpret=False):
    M = xyzf.shape[0]
    S = M // _B
    (w1x, w1p, b1, g1, be1, w2, b2, g2, be2, w3, b3, g3, be3,
     l1w, l1b, l1g, l1be, l2w, l2b, l2g, l2be, l3w, l3b) = range(23)

    def body(*refs):
        x_ref, p_ref = refs[0], refs[1]
        a_ref, c_ref = refs[2], refs[3]
        W = [refs[4 + i] for i in range(23)]
        lo_ref, l3_ref = refs[27], refs[28]

        def bnrelu(y, g, be):
            m = jnp.mean(y, axis=0, keepdims=True)
            v = jnp.mean((y - m) ** 2, axis=0, keepdims=True)
            return jnp.maximum((y - m) / jnp.sqrt(v + _EPS) * g + be, 0.0)

        def mm(a, b):
            return lax.dot_general(a, b, (((1,), (0,)), ((), ())),
                                   preferred_element_type=jnp.float32)

        pts = jnp.maximum(p_ref[...] * a_ref[0][None, :] + c_ref[0][None, :],
                          0.0)
        y = mm(x_ref[...], W[w1x][...]) + mm(pts, W[w1p][...]) \
            + W[b1][0][None, :]
        y = bnrelu(y, W[g1][0][None, :], W[be1][0][None, :])
        y = bnrelu(mm(y, W[w2][...]) + W[b2][0][None, :],
                   W[g2][0][None, :], W[be2][0][None, :])
        y = mm(y, W[w3][...]) + W[b3][0][None, :]
        m = jnp.mean(y, axis=0, keepdims=True)
        v = jnp.mean((y - m) ** 2, axis=0, keepdims=True)
        y = jnp.maximum((y - m) / jnp.sqrt(v + _EPS) * W[g3][0][None, :]
                        + W[be3][0][None, :], 0.0)
        l3 = jnp.max(y.reshape(_B, S, y.shape[1]), axis=1)
        l3_ref[...] = l3

        def bnrelu_b(y, g, be):
            m = jnp.mean(y, axis=0, keepdims=True)
            v = jnp.mean((y - m) ** 2, axis=0, keepdims=True)
            return jnp.maximum((y - m) / jnp.sqrt(v + _EPS) * g + be, 0.0)

        h = bnrelu_b(mm(l3, W[l1w][...]) + W[l1b][0][None, :],
                     W[l1g][0][None, :], W[l1be][0][None, :])
        h = bnrelu_b(mm(h, W[l2w][...]) + W[l2b][0][None, :],
                     W[l2g][0][None, :], W[l2be][0][None, :])
        h = mm(h, W[l3w][...]) + W[l3b][0][None, :]
        mx = jnp.max(h, axis=1, keepdims=True)
        sh = h - mx
        lo_ref[...] = sh - jnp.log(jnp.sum(jnp.exp(sh), axis=1, keepdims=True))

    return pl.pallas_call(
        body,
        out_shape=[jax.ShapeDtypeStruct((_B, 40), jnp.float32),
                   jax.ShapeDtypeStruct((_B, 1024), jnp.float32)],
        interpret=interpret,
    )(xyzf, P2, a6, c6, *w)


def _affine(sums, count, g, be):
    m = sums[0] / count
    v = jnp.maximum(sums[1] / count - m * m, 0.0)
    inv = 1.0 / jnp.sqrt(v + _EPS)
    a = g * inv
    c = be - m * a
    return a[None], c[None]


def _row(v):
    return v[None, :]


def _sa_stage(coords, npoint, r2, K, table, Wx, Wf, layers, interpret=False):
    Bb, _, N = coords.shape
    fidx = _fps(coords, npoint, interpret)
    cen = _cgather(coords, fidx, interpret)
    gidx = _ball(coords, cen, r2, K, interpret)
    gflat = jnp.transpose(gidx, (0, 2, 1)).reshape(-1)
    ct = jnp.transpose(cen, (0, 2, 1)).reshape(Bb * npoint, 3)

    G = _sc_gather(table, gflat)

    (l1, l2, l3) = layers
    CP = table.shape[1]
    BS = Bb * npoint
    M = BS * K
    Gr = G.reshape(BS, K, CP)

    s1 = _stats(Gr, ct, Wx, Wf, _row(l1["b"]), interpret)
    a1, c1 = _affine(s1, M, l1["g"], l1["be"])
    y2, s2 = _mid(Gr, ct, Wx, Wf, _row(l1["b"]), a1, c1, l2["W"],
                  _row(l2["b"]), interpret)
    a2, c2 = _affine(s2, M, l2["g"], l2["be"])
    P, s3 = _lastpool(y2, a2, c2, l3["W"], _row(l3["b"]), interpret)
    a3, c3 = _affine(s3, M, l3["g"], l3["be"])
    return cen, P, a3, c3


def kernel(x, params):
    Bb, N, _ = x.shape
    coords = jnp.transpose(x, (0, 2, 1))

    sa1 = params["sa1"]
    t1 = jnp.pad(x.reshape(Bb * N, 3), ((0, 0), (0, 13)))
    c1, P1, a3, c3 = _sa_stage(coords, 512, float(0.2 ** 2), 32, t1,
                               sa1[0]["W"], None, sa1)

    sa2 = params["sa2"]
    W4 = sa2[0]["W"]
    T2 = _table2(c1, P1.reshape(Bb, 512, 128), a3, c3)
    c2, P2, a6, c6 = _sa_stage(c1, 256, float(0.4 ** 2), 64,
                               T2.reshape(Bb * 512, 144), W4[:3], W4[3:], sa2)

    sa3 = params["sa3"]
    W7 = sa3[0]["W"]
    xyzf = jnp.transpose(c2, (0, 2, 1)).reshape(Bb * 256, 3)
    w = [W7[:3], W7[3:], _row(sa3[0]["b"]), _row(sa3[0]["g"]),
         _row(sa3[0]["be"]),
         sa3[1]["W"], _row(sa3[1]["b"]), _row(sa3[1]["g"]), _row(sa3[1]["be"]),
         sa3[2]["W"], _row(sa3[2]["b"]), _row(sa3[2]["g"]), _row(sa3[2]["be"]),
         params["lin1"]["W"], _row(params["lin1"]["b"]),
         _row(params["lin1"]["g"]), _row(params["lin1"]["be"]),
         params["lin2"]["W"], _row(params["lin2"]["b"]),
         _row(params["lin2"]["g"]), _row(params["lin2"]["be"]),
         params["lin3"]["W"], _row(params["lin3"]["b"])]
    logits, l3 = _sa3_head(xyzf, P2, a6, c6, w)
    return logits, l3.reshape(Bb, 1, 1024)
